# Initial kernel scaffold; baseline (speedup 1.0000x reference)
import jax
import jax.numpy as jnp
from jax.experimental import pallas as pl


def kernel(h_qa, A_uq, u_embed, Wu, Wq, bu, bq):
    raise NotImplementedError("write your pallas kernel here")


# speedup vs baseline: 8.0146x; 8.0146x over previous
import functools

import jax
import jax.numpy as jnp
from jax import lax
from jax.experimental import pallas as pl
from jax.experimental.pallas import tpu as pltpu
from jax.experimental.pallas import tpu_sc as plsc

N = 10000
D = 128
E = 320000
NLAYERS = 3

NC = 2
NS = 16
NW = NC * NS
CH = 128
CPT = 79
EPT = CPT * CH
EPAD = EPT * NW
NROWS2D = EPAD // CH
DCPT = NROWS2D // NS

NPAD = 10240
RPT = NPAD // NS
ZR = 64

_mesh = plsc.VectorSubcoreMesh(core_axis_name="c", subcore_axis_name="s")


def _sc_pass_body(g_hbm, src_hbm, dst_hbm, z_hbm, out_hbm,
                  src_v, dst_v, rows_v, zbuf, acc, sem):
    c = lax.axis_index("c")
    s = lax.axis_index("s")
    wid = c * NS + s
    pltpu.sync_copy(src_hbm.at[wid], src_v)
    pltpu.sync_copy(dst_hbm.at[wid], dst_v)
    pltpu.sync_copy(z_hbm, zbuf)
    for k in range(RPT // ZR):
        pltpu.sync_copy(zbuf, acc.at[pl.ds(s * RPT + k * ZR, ZR)])
    plsc.subcore_barrier()

    def body(j, t):
        pltpu.async_copy(g_hbm.at[src_v.at[j]], rows_v, sem).wait()
        pltpu.sync_copy(rows_v, acc.at[dst_v.at[j]], add=True)
        return t

    lax.fori_loop(0, CPT, body, 0)
    plsc.subcore_barrier()
    for k in range(RPT // ZR):
        r0 = s * RPT + k * ZR
        pltpu.sync_copy(acc.at[pl.ds(r0, ZR)], zbuf)
        pltpu.sync_copy(zbuf, out_hbm.at[c, pl.ds(r0, ZR)])


_sc_pass = pl.kernel(
    _sc_pass_body, mesh=_mesh,
    out_type=jax.ShapeDtypeStruct((NC, NPAD, D), jnp.float32),
    scratch_types=[
        pltpu.VMEM((CPT, CH), jnp.int32),
        pltpu.VMEM((CPT, CH), jnp.int32),
        pltpu.VMEM((CH, D), jnp.float32),
        pltpu.VMEM((ZR, D), jnp.float32),
        pltpu.VMEM_SHARED((NPAD, D), jnp.float32),
        pltpu.SemaphoreType.DMA,
    ],
)


def _sc_deg_body(uq_hbm, z1_hbm, o1_hbm, out_hbm,
                 idx_v, ones_v, zb1, acc1):
    c = lax.axis_index("c")
    s = lax.axis_index("s")
    pltpu.sync_copy(uq_hbm.at[c, s], idx_v)
    pltpu.sync_copy(o1_hbm, ones_v)
    pltpu.sync_copy(z1_hbm, zb1)
    pltpu.sync_copy(zb1, acc1.at[pl.ds(s * RPT, RPT)])
    plsc.subcore_barrier()

    def body(j, t):
        pltpu.sync_copy(ones_v, acc1.at[idx_v.at[j]], add=True)
        return t

    lax.fori_loop(0, DCPT, body, 0)
    plsc.subcore_barrier()
    pltpu.sync_copy(acc1.at[pl.ds(s * RPT, RPT)], zb1)
    pltpu.sync_copy(zb1, out_hbm.at[pl.ds(c * NPAD + s * RPT, RPT)])


_sc_deg = pl.kernel(
    _sc_deg_body, mesh=_mesh,
    out_type=jax.ShapeDtypeStruct((NC * NPAD,), jnp.float32),
    scratch_types=[
        pltpu.VMEM((DCPT, CH), jnp.int32),
        pltpu.VMEM((CH,), jnp.float32),
        pltpu.VMEM((RPT,), jnp.float32),
        pltpu.VMEM_SHARED((NPAD,), jnp.float32),
    ],
)

BR = 512


def _prep_body(du_ref, dq_ref, hq_ref, ue_ref, gq_ref, gu_ref):
    ru = lax.rsqrt(jnp.maximum(du_ref[...], 1.0))
    rq = lax.rsqrt(jnp.maximum(dq_ref[...], 1.0))
    gq_ref[...] = rq * hq_ref[...]
    gu_ref[...] = ru * ue_ref[...]


_prep = pl.pallas_call(
    _prep_body,
    grid=(NPAD // BR,),
    in_specs=[pl.BlockSpec((BR, D), lambda i: (i, 0))] * 4,
    out_specs=[pl.BlockSpec((BR, D), lambda i: (i, 0))] * 2,
    out_shape=[jax.ShapeDtypeStruct((NPAD, D), jnp.float32)] * 2,
)


def _layer_body(p_ref, deg_ref, w_ref, b_ref, o_ref, *, relu, rescale):
    r = lax.rsqrt(jnp.maximum(deg_ref[...], 1.0))
    m = (p_ref[0] + p_ref[1]) * r
    h = jnp.dot(m, w_ref[...], preferred_element_type=jnp.float32) + b_ref[...]
    if relu:
        h = jnp.maximum(h, 0.0)
    if rescale:
        h = h * r
    o_ref[...] = h


def _make_layer(relu, rescale):
    return pl.pallas_call(
        functools.partial(_layer_body, relu=relu, rescale=rescale),
        grid=(NPAD // BR,),
        in_specs=[
            pl.BlockSpec((NC, BR, D), lambda i: (0, i, 0)),
            pl.BlockSpec((BR, D), lambda i: (i, 0)),
            pl.BlockSpec((D, D), lambda i: (0, 0)),
            pl.BlockSpec((1, D), lambda i: (0, 0)),
        ],
        out_specs=pl.BlockSpec((BR, D), lambda i: (i, 0)),
        out_shape=jax.ShapeDtypeStruct((NPAD, D), jnp.float32),
    )


_layer_mid = _make_layer(relu=True, rescale=True)
_layer_last = _make_layer(relu=False, rescale=False)


def kernel(h_qa, A_uq, u_embed, Wu, Wq, bu, bq):
    ii = A_uq.astype(jnp.int32)
    pad = jnp.full((EPAD - E,), N, jnp.int32)
    u2 = jnp.concatenate([ii[0], pad]).reshape(NW, CPT, CH)
    q2 = jnp.concatenate([ii[1], pad]).reshape(NW, CPT, CH)
    uq = jnp.stack([u2, q2]).reshape(NC, NS, DCPT, CH)
    z2 = jnp.zeros((ZR, D), jnp.float32)
    z1 = jnp.zeros((RPT,), jnp.float32)
    o1 = jnp.ones((CH,), jnp.float32)

    deg = _sc_deg(uq, z1, o1).reshape(NC, NPAD)
    DEGu = jnp.broadcast_to(deg[0][:, None], (NPAD, D))
    DEGq = jnp.broadcast_to(deg[1][:, None], (NPAD, D))
    hq_p = jnp.pad(h_qa, ((0, NPAD - N), (0, 0)))
    ue_p = jnp.pad(u_embed, ((0, NPAD - N), (0, 0)))
    gq, gu = _prep(DEGu, DEGq, hq_p, ue_p)

    for l in range(NLAYERS):
        last = l == NLAYERS - 1
        pq = _sc_pass(gu, u2, q2, z2)
        if not last:
            pu = _sc_pass(gq, q2, u2, z2)
            gu = _layer_mid(pu, DEGu, Wu[l], bu[l].reshape(1, D))
            gq = _layer_mid(pq, DEGq, Wq[l], bq[l].reshape(1, D))
        else:
            hq_out = _layer_last(pq, DEGq, Wq[l], bq[l].reshape(1, D))
    return hq_out[:N]
